# baseline (device time: 23765 ns/iter reference)
import jax
import jax.numpy as jnp
from jax import lax
from jax.experimental import pallas as pl
from jax.experimental.pallas import tpu as pltpu

N_DEV = 8


def kernel(x, w_mat):
    m_per, k = x.shape
    _, n_per = w_mat.shape

    def body(
        x_ref,
        w_ref,
        out_ref,
        xf32_ref,
        wf32_ref,
        xfull_ref,
        wb_ref,
        local_sems,
        send_sems,
        recv_sems,
    ):
        my = lax.axis_index("i")

        cpx = pltpu.make_async_copy(x_ref, xf32_ref, local_sems.at[0])
        cpx.start()
        cpw = pltpu.make_async_copy(w_ref, wf32_ref, local_sems.at[1])
        cpw.start()

        barrier_sem = pltpu.get_barrier_semaphore()
        for off in range(1, N_DEV):
            pl.semaphore_signal(
                barrier_sem,
                inc=1,
                device_id=(lax.rem(my + off, N_DEV),),
                device_id_type=pl.DeviceIdType.MESH,
            )
        pl.semaphore_wait(barrier_sem, N_DEV - 1)

        cpx.wait()
        xfull_ref[pl.ds(my * m_per, m_per), :] = xf32_ref[...].astype(
            jnp.bfloat16
        )

        sends = []
        for off in range(1, N_DEV):
            tgt = lax.rem(my + off, N_DEV)
            rdma = pltpu.make_async_remote_copy(
                src_ref=xfull_ref.at[pl.ds(my * m_per, m_per), :],
                dst_ref=xfull_ref.at[pl.ds(my * m_per, m_per), :],
                send_sem=send_sems.at[off],
                recv_sem=recv_sems.at[off],
                device_id=(tgt,),
                device_id_type=pl.DeviceIdType.MESH,
            )
            rdma.start()
            sends.append(rdma)

        cpw.wait()
        wb_ref[...] = wf32_ref[...].astype(jnp.bfloat16)

        def block_gemm(origin):
            chunk = xfull_ref[pl.ds(origin * m_per, m_per), :]
            acc = jnp.dot(
                chunk, wb_ref[...], preferred_element_type=jnp.float32
            )
            out_ref[pl.ds(origin * m_per, m_per), :] = jnp.maximum(acc, 0.0)

        block_gemm(my)

        for off in range(1, N_DEV):
            origin = lax.rem(my - off + N_DEV, N_DEV)
            recv = pltpu.make_async_remote_copy(
                src_ref=xfull_ref.at[pl.ds(origin * m_per, m_per), :],
                dst_ref=xfull_ref.at[pl.ds(origin * m_per, m_per), :],
                send_sem=send_sems.at[0],
                recv_sem=recv_sems.at[off],
                device_id=(my,),
                device_id_type=pl.DeviceIdType.MESH,
            )
            recv.wait_recv()
            block_gemm(origin)

        for rdma in sends:
            rdma.wait_send()

    return pl.pallas_call(
        body,
        out_shape=jax.ShapeDtypeStruct((N_DEV * m_per, n_per), jnp.float32),
        in_specs=[
            pl.BlockSpec(memory_space=pl.ANY),
            pl.BlockSpec(memory_space=pl.ANY),
        ],
        out_specs=pl.BlockSpec(memory_space=pltpu.VMEM),
        scratch_shapes=[
            pltpu.VMEM((m_per, k), jnp.float32),
            pltpu.VMEM((k, n_per), jnp.float32),
            pltpu.VMEM((N_DEV * m_per, k), jnp.bfloat16),
            pltpu.VMEM((k, n_per), jnp.bfloat16),
            pltpu.SemaphoreType.DMA((2,)),
            pltpu.SemaphoreType.DMA((N_DEV,)),
            pltpu.SemaphoreType.DMA((N_DEV,)),
        ],
        compiler_params=pltpu.CompilerParams(collective_id=0),
    )(x, w_mat)
